# Initial kernel scaffold; baseline (speedup 1.0000x reference)
import jax
import jax.numpy as jnp
from jax.experimental import pallas as pl


def kernel(t, h, W, b):
    raise NotImplementedError("write your pallas kernel here")


# speedup vs baseline: 24.4579x; 24.4579x over previous
import jax
import jax.numpy as jnp
import numpy as np
from jax.experimental import pallas as pl

N_NODES = 100000
DIM = 128
ROWS = 4000

_ADJ = np.array([[0, 1, 1, 1],
                 [1, 0, 0, 0],
                 [1, 0, 0, 1],
                 [1, 0, 1, 0]], dtype=np.float32)

_DEG = _ADJ.sum(axis=0) + 1.0
_DIS = 1.0 / np.sqrt(_DEG)
_S4 = (_ADJ.T + np.eye(4, dtype=np.float32)) * np.outer(_DIS, _DIS)

_S128 = np.eye(DIM, dtype=np.float32)
_S128[:4, :4] = _S4


def _body(h_ref, W_ref, b_ref, S_ref, out_ref):
    x = jnp.dot(h_ref[...], W_ref[...], preferred_element_type=jnp.float32)
    out_ref[...] = jnp.tanh(x + b_ref[...])

    @pl.when(pl.program_id(0) == 0)
    def _():
        top = jnp.dot(S_ref[...], x[:DIM, :], preferred_element_type=jnp.float32)
        out_ref[:DIM, :] = jnp.tanh(top + b_ref[...])


def kernel(t, h, W, b):
    del t
    b2 = b.reshape(1, DIM)
    S = jnp.asarray(_S128)
    return pl.pallas_call(
        _body,
        grid=(N_NODES // ROWS,),
        in_specs=[
            pl.BlockSpec((ROWS, DIM), lambda i: (i, 0)),
            pl.BlockSpec((DIM, DIM), lambda i: (0, 0)),
            pl.BlockSpec((1, DIM), lambda i: (0, 0)),
            pl.BlockSpec((DIM, DIM), lambda i: (0, 0)),
        ],
        out_specs=pl.BlockSpec((ROWS, DIM), lambda i: (i, 0)),
        out_shape=jax.ShapeDtypeStruct((N_NODES, DIM), jnp.float32),
    )(h, W, b2, S)


# speedup vs baseline: 28.5522x; 1.1674x over previous
import jax
import jax.numpy as jnp
import numpy as np
from jax.experimental import pallas as pl

N_NODES = 100000
DIM = 128
ROWS = 10000

_ADJ = np.array([[0, 1, 1, 1],
                 [1, 0, 0, 0],
                 [1, 0, 0, 1],
                 [1, 0, 1, 0]], dtype=np.float32)

_DEG = _ADJ.sum(axis=0) + 1.0
_DIS = 1.0 / np.sqrt(_DEG)
_S4 = (_ADJ.T + np.eye(4, dtype=np.float32)) * np.outer(_DIS, _DIS)

_S128 = np.eye(DIM, dtype=np.float32)
_S128[:4, :4] = _S4


def _body(h_ref, W_ref, b_ref, S_ref, out_ref):
    x = jnp.dot(h_ref[...], W_ref[...], preferred_element_type=jnp.float32)
    out_ref[...] = jnp.tanh(x + b_ref[...])

    @pl.when(pl.program_id(0) == 0)
    def _():
        top = jnp.dot(S_ref[...], x[:DIM, :], preferred_element_type=jnp.float32)
        out_ref[:DIM, :] = jnp.tanh(top + b_ref[...])


def kernel(t, h, W, b):
    del t
    b2 = b.reshape(1, DIM)
    S = jnp.asarray(_S128)
    return pl.pallas_call(
        _body,
        grid=(N_NODES // ROWS,),
        in_specs=[
            pl.BlockSpec((ROWS, DIM), lambda i: (i, 0)),
            pl.BlockSpec((DIM, DIM), lambda i: (0, 0)),
            pl.BlockSpec((1, DIM), lambda i: (0, 0)),
            pl.BlockSpec((DIM, DIM), lambda i: (0, 0)),
        ],
        out_specs=pl.BlockSpec((ROWS, DIM), lambda i: (i, 0)),
        out_shape=jax.ShapeDtypeStruct((N_NODES, DIM), jnp.float32),
    )(h, W, b2, S)


# speedup vs baseline: 29.8801x; 1.0465x over previous
import jax
import jax.numpy as jnp
import numpy as np
from jax.experimental import pallas as pl

N_NODES = 100000
DIM = 128
ROWS = 20000

_ADJ = np.array([[0, 1, 1, 1],
                 [1, 0, 0, 0],
                 [1, 0, 0, 1],
                 [1, 0, 1, 0]], dtype=np.float32)

_DEG = _ADJ.sum(axis=0) + 1.0
_DIS = 1.0 / np.sqrt(_DEG)
_S4 = (_ADJ.T + np.eye(4, dtype=np.float32)) * np.outer(_DIS, _DIS)

_S128 = np.eye(DIM, dtype=np.float32)
_S128[:4, :4] = _S4


def _body(h_ref, W_ref, b_ref, S_ref, out_ref):
    x = jnp.dot(h_ref[...], W_ref[...], preferred_element_type=jnp.float32)
    out_ref[...] = jnp.tanh(x + b_ref[...])

    @pl.when(pl.program_id(0) == 0)
    def _():
        top = jnp.dot(S_ref[...], x[:DIM, :], preferred_element_type=jnp.float32)
        out_ref[:DIM, :] = jnp.tanh(top + b_ref[...])


def kernel(t, h, W, b):
    del t
    b2 = b.reshape(1, DIM)
    S = jnp.asarray(_S128)
    return pl.pallas_call(
        _body,
        grid=(N_NODES // ROWS,),
        in_specs=[
            pl.BlockSpec((ROWS, DIM), lambda i: (i, 0)),
            pl.BlockSpec((DIM, DIM), lambda i: (0, 0)),
            pl.BlockSpec((1, DIM), lambda i: (0, 0)),
            pl.BlockSpec((DIM, DIM), lambda i: (0, 0)),
        ],
        out_specs=pl.BlockSpec((ROWS, DIM), lambda i: (i, 0)),
        out_shape=jax.ShapeDtypeStruct((N_NODES, DIM), jnp.float32),
    )(h, W, b2, S)
